# baseline (speedup 1.0000x reference)
import jax
import jax.numpy as jnp
from jax import lax
from jax.experimental import pallas as pl
from jax.experimental.pallas import tpu as pltpu
from jax.experimental.pallas import tpu_sc as plsc

S = 64.0
M = 0.4

_W = 2048
_KCHUNKS = 48
_SC_COLS = _W * _KCHUNKS
_SC_ROWS = 512
_RG = 8
_NBUF = 3
_VPC = _W // 16
_UNROLL = 16
_TC_ROWBLK = 16
_TC_BLK = 128


def _sc_body(x_hbm, cmp_hbm, o_hbm, inbuf, outbuf, cmpbuf, insem, outsem):
    info = plsc.get_sparse_core_info()
    nc = info.num_cores
    wid = lax.axis_index("s") * nc + lax.axis_index("c")
    n_groups = _SC_ROWS // _RG // 32
    rows_per_worker = _SC_ROWS // 32
    n_chunks = n_groups * _KCHUNKS

    pltpu.sync_copy(cmp_hbm.at[pl.ds(n_groups * wid, n_groups), :, :], cmpbuf)

    def row0(t):
        return rows_per_worker * wid + _RG * (t // _KCHUNKS)

    def col0(t):
        return _W * lax.rem(t, _KCHUNKS)

    def in_cp(t, s):
        return pltpu.make_async_copy(
            x_hbm.at[pl.ds(row0(t), _RG), pl.ds(col0(t), _W)],
            inbuf.at[s], insem.at[s])

    def out_cp(t, s):
        return pltpu.make_async_copy(
            outbuf.at[s], o_hbm.at[pl.ds(row0(t), _RG), pl.ds(col0(t), _W)],
            outsem.at[s])

    for s in range(_NBUF):
        in_cp(s, s).start()

    scale = jnp.float32(S)
    marg = jnp.float32(M)
    zero = jnp.float32(0.0)

    def step(t, carry):
        s = lax.rem(t, _NBUF)
        in_cp(t, s).wait()

        @pl.when(t >= _NBUF)
        def _():
            out_cp(t - _NBUF, s).wait()

        g = t // _KCHUNKS
        base_gv = lax.rem(t, _KCHUNKS) * _VPC

        for r in range(_RG):
            cr = cmpbuf[g, r, :]

            @plsc.parallel_loop(0, _VPC, unroll=_UNROLL)
            def _(j, s=s, r=r, cr=cr):
                v = inbuf[s, r, pl.ds(j * 16, 16)]
                m = cr == base_gv + j
                outbuf[s, r, pl.ds(j * 16, 16)] = (
                    v - jnp.where(m, marg, zero)) * scale

        out_cp(t, s).start()

        @pl.when(t + _NBUF < n_chunks)
        def _():
            in_cp(t + _NBUF, s).start()

        return carry

    lax.fori_loop(0, n_chunks, step, 0)

    for s in range(_NBUF):
        out_cp(n_chunks - _NBUF + s, s).wait()


def _tc_a_body(lab_ref, x_ref, o_ref):
    x = x_ref[...]
    lab = lab_ref[...]
    cols = jax.lax.broadcasted_iota(jnp.int32, x.shape, 1)
    margin = jnp.where(cols == lab, jnp.float32(-M), jnp.float32(0.0))
    o_ref[...] = (x + margin) * jnp.float32(S)


def _tc_b_body(lab_ref, x_ref, alias_ref, o_ref):
    j = pl.program_id(0)
    x = x_ref[...]
    lab = lab_ref[...]
    cols = (jax.lax.broadcasted_iota(jnp.int32, x.shape, 1)
            + _SC_COLS + j * _TC_BLK)
    margin = jnp.where(cols == lab, jnp.float32(-M), jnp.float32(0.0))
    o_ref[...] = (x + margin) * jnp.float32(S)


def kernel(logits, labels):
    B, C = logits.shape
    lab = labels.astype(jnp.int32)
    valid = lab != -1
    top = lab[:_SC_ROWS]
    in_sc = valid[:_SC_ROWS] & (top < _SC_COLS)
    gv = jnp.where(in_sc, top // 16, -1)
    lanepos = jnp.where(in_sc, top % 16, -1)
    cmp = jnp.where(jnp.arange(16, dtype=jnp.int32)[None, :] == lanepos[:, None],
                    gv[:, None], -1)
    ngrp = _SC_ROWS // _RG
    cmp3 = cmp.reshape(ngrp, _RG, 16)

    mesh = plsc.VectorSubcoreMesh(core_axis_name="c", subcore_axis_name="s")
    sc_run = pl.kernel(
        _sc_body,
        mesh=mesh,
        out_type=jax.ShapeDtypeStruct((_SC_ROWS, _SC_COLS), logits.dtype),
        scratch_types=[
            pltpu.VMEM((_NBUF, _RG, _W), logits.dtype),
            pltpu.VMEM((_NBUF, _RG, _W), logits.dtype),
            pltpu.VMEM((ngrp // 32, _RG, 16), jnp.int32),
            pltpu.SemaphoreType.DMA((_NBUF,)),
            pltpu.SemaphoreType.DMA((_NBUF,)),
        ],
    )
    sc_out = sc_run(logits, cmp3)

    lab2d = lab.reshape(B, 1)
    row_base = _SC_ROWS // _TC_ROWBLK
    tc_a = pl.pallas_call(
        _tc_a_body,
        grid=((B - _SC_ROWS) // _TC_ROWBLK,),
        in_specs=[
            pl.BlockSpec((_TC_ROWBLK, 1), lambda i: (row_base + i, 0)),
            pl.BlockSpec((_TC_ROWBLK, C), lambda i: (row_base + i, 0)),
        ],
        out_specs=pl.BlockSpec((_TC_ROWBLK, C), lambda i: (row_base + i, 0)),
        out_shape=jax.ShapeDtypeStruct((B, C), logits.dtype),
        compiler_params=pltpu.CompilerParams(
            dimension_semantics=("arbitrary",),
        ),
    )(lab2d, logits)

    tail_blocks = -(-(C - _SC_COLS) // _TC_BLK)
    col_base = _SC_COLS // _TC_BLK
    tc_b = pl.pallas_call(
        _tc_b_body,
        grid=(tail_blocks,),
        in_specs=[
            pl.BlockSpec((_SC_ROWS, 1), lambda j: (0, 0)),
            pl.BlockSpec((_SC_ROWS, _TC_BLK), lambda j: (0, col_base + j)),
            pl.BlockSpec(memory_space=pl.ANY),
        ],
        out_specs=pl.BlockSpec((_SC_ROWS, _TC_BLK), lambda j: (0, col_base + j)),
        out_shape=jax.ShapeDtypeStruct((B, C), logits.dtype),
        input_output_aliases={2: 0},
        compiler_params=pltpu.CompilerParams(
            dimension_semantics=("arbitrary",),
        ),
    )(lab2d, logits, tc_a)
    return lax.dynamic_update_slice(tc_b, sc_out, (0, 0))


# speedup vs baseline: 1.0921x; 1.0921x over previous
import jax
import jax.numpy as jnp
from jax.experimental import pallas as pl
from jax.experimental.pallas import tpu as pltpu

S = 64.0
M = 0.4

_ROWS = 16
_HALF = 8
_DEPTH = 4


def _body(lab_ref, x_hbm, o_hbm, xbuf, obuf, insem_a, insem_b,
          outsem_a, outsem_b):
    n_chunks = x_hbm.shape[0] // _ROWS

    def in_a(i, s):
        return pltpu.make_async_copy(
            x_hbm.at[pl.ds(i * _ROWS, _HALF), :],
            xbuf.at[s, pl.ds(0, _HALF)], insem_a.at[s])

    def in_b(i, s):
        return pltpu.make_async_copy(
            x_hbm.at[pl.ds(i * _ROWS + _HALF, _HALF), :],
            xbuf.at[s, pl.ds(_HALF, _HALF)], insem_b.at[s])

    def out_a(i, s):
        return pltpu.make_async_copy(
            obuf.at[s, pl.ds(0, _HALF)],
            o_hbm.at[pl.ds(i * _ROWS, _HALF), :], outsem_a.at[s])

    def out_b(i, s):
        return pltpu.make_async_copy(
            obuf.at[s, pl.ds(_HALF, _HALF)],
            o_hbm.at[pl.ds(i * _ROWS + _HALF, _HALF), :], outsem_b.at[s])

    for s in range(_DEPTH):
        in_a(s, s).start()
        in_b(s, s).start()

    def step(i, carry):
        s = jax.lax.rem(i, _DEPTH)
        in_a(i, s).wait()
        in_b(i, s).wait()

        @pl.when(i >= _DEPTH)
        def _():
            out_a(i - _DEPTH, s).wait()
            out_b(i - _DEPTH, s).wait()

        x = xbuf[s]
        lab = lab_ref[pl.ds(i * _ROWS, _ROWS), :]
        cols = jax.lax.broadcasted_iota(jnp.int32, x.shape, 1)
        margin = jnp.where(cols == lab, jnp.float32(-M), jnp.float32(0.0))
        obuf[s] = (x + margin) * jnp.float32(S)

        out_a(i, s).start()
        out_b(i, s).start()

        @pl.when(i + _DEPTH < n_chunks)
        def _():
            in_a(i + _DEPTH, s).start()
            in_b(i + _DEPTH, s).start()

        return carry

    jax.lax.fori_loop(0, n_chunks, step, 0)

    for s in range(_DEPTH):
        out_a(n_chunks - _DEPTH + s, s).wait()
        out_b(n_chunks - _DEPTH + s, s).wait()


def kernel(logits, labels):
    B, C = logits.shape
    lab2d = labels.astype(jnp.int32).reshape(B, 1)
    return pl.pallas_call(
        _body,
        in_specs=[
            pl.BlockSpec(memory_space=pltpu.VMEM),
            pl.BlockSpec(memory_space=pl.ANY),
        ],
        out_specs=pl.BlockSpec(memory_space=pl.ANY),
        out_shape=jax.ShapeDtypeStruct((B, C), logits.dtype),
        scratch_shapes=[
            pltpu.VMEM((_DEPTH, _ROWS, C), logits.dtype),
            pltpu.VMEM((_DEPTH, _ROWS, C), logits.dtype),
            pltpu.SemaphoreType.DMA((_DEPTH,)),
            pltpu.SemaphoreType.DMA((_DEPTH,)),
            pltpu.SemaphoreType.DMA((_DEPTH,)),
            pltpu.SemaphoreType.DMA((_DEPTH,)),
        ],
    )(lab2d, logits)
